# Initial kernel scaffold; baseline (speedup 1.0000x reference)
import jax
import jax.numpy as jnp
from jax.experimental import pallas as pl


def kernel(x, tables):
    raise NotImplementedError("write your pallas kernel here")


# speedup vs baseline: 1.1852x; 1.1852x over previous
import jax
import jax.numpy as jnp
from jax import lax
from jax.experimental import pallas as pl
from jax.experimental.pallas import tpu as pltpu
from jax.experimental.pallas import tpu_sc as plsc

_NUM_FIELDS = 26
_VOCAB = 100000
_D = 32
_BATCH = 16384
_NC, _NS = 2, 16
_NW = _NC * _NS
_TOTAL = _NUM_FIELDS * _BATCH
_PER_W = _TOTAL // _NW
_IDX_W = 128
_IDXROWS_W = _PER_W // _IDX_W
_G = 8
_CHUNK = _G * _IDX_W
_NCHUNK = _IDXROWS_W // _G


def _body(idx_hbm, table_hbm, out_hbm, idx_v, rows_v, sem):
    wid = lax.axis_index("s") * _NC + lax.axis_index("c")
    irow0 = wid * _IDXROWS_W
    row0 = wid * _PER_W
    pltpu.sync_copy(idx_hbm.at[pl.ds(irow0, _IDXROWS_W)], idx_v)

    def chunk(c, carry):
        cps = [
            pltpu.async_copy(
                table_hbm.at[idx_v.at[c * _G + j]],
                rows_v.at[pl.ds(j * _IDX_W, _IDX_W)],
                sem,
            )
            for j in range(_G)
        ]
        for cp in cps:
            cp.wait()
        pltpu.sync_copy(rows_v, out_hbm.at[pl.ds(row0 + c * _CHUNK, _CHUNK)])
        return carry

    lax.fori_loop(0, _NCHUNK, chunk, 0)


@jax.jit
def _run(idx, table_flat):
    k = pl.kernel(
        _body,
        out_type=jax.ShapeDtypeStruct((_TOTAL, _D), jnp.float32),
        mesh=plsc.VectorSubcoreMesh(core_axis_name="c", subcore_axis_name="s"),
        scratch_types=[
            pltpu.VMEM((_IDXROWS_W, _IDX_W), jnp.int32),
            pltpu.VMEM((_CHUNK, _D), jnp.float32),
            pltpu.SemaphoreType.DMA,
        ],
        compiler_params=pltpu.CompilerParams(use_tc_tiling_on_sc=False),
    )
    return k(idx, table_flat)


def kernel(x, tables):
    offs = (jnp.arange(_NUM_FIELDS, dtype=jnp.int32) * _VOCAB)[:, None]
    idx = (x.T + offs).reshape(_TOTAL // _IDX_W, _IDX_W)
    out = _run(idx, tables.reshape(_NUM_FIELDS * _VOCAB, _D))
    return out.reshape(_NUM_FIELDS, _BATCH, _D)


# speedup vs baseline: 1.1898x; 1.0039x over previous
import jax
import jax.numpy as jnp
from jax import lax
from jax.experimental import pallas as pl
from jax.experimental.pallas import tpu as pltpu
from jax.experimental.pallas import tpu_sc as plsc

_NUM_FIELDS = 26
_VOCAB = 100000
_D = 32
_BATCH = 16384
_NC, _NS = 2, 16
_NW = _NC * _NS
_TOTAL = _NUM_FIELDS * _BATCH
_PER_W = _TOTAL // _NW
_IDX_W = 128
_IDXROWS_W = _PER_W // _IDX_W
_G = 8
_CHUNK = _G * _IDX_W
_NCHUNK = _IDXROWS_W // _G
_NBUF = 3


def _body(idx_hbm, table_hbm, out_hbm, idx_v, rows_v, gsems, osems):
    wid = lax.axis_index("s") * _NC + lax.axis_index("c")
    irow0 = wid * _IDXROWS_W
    row0 = wid * _PER_W
    pltpu.sync_copy(idx_hbm.at[pl.ds(irow0, _IDXROWS_W)], idx_v)

    def fire_gathers(q, b):
        return [
            pltpu.async_copy(
                table_hbm.at[idx_v.at[q * _G + j]],
                rows_v.at[b].at[pl.ds(j * _IDX_W, _IDX_W)],
                gsems[b],
            )
            for j in range(_G)
        ]

    gather_cps = {}
    wb_cps = {}
    for q in range(_NCHUNK + 1):
        if q < _NCHUNK:
            b = q % _NBUF
            if q >= _NBUF:
                wb_cps.pop(q - _NBUF).wait()
            gather_cps[q] = fire_gathers(q, b)
        if q >= 1:
            qq = q - 1
            bb = qq % _NBUF
            for cp in gather_cps.pop(qq):
                cp.wait()
            wb_cps[qq] = pltpu.async_copy(
                rows_v.at[bb],
                out_hbm.at[pl.ds(row0 + qq * _CHUNK, _CHUNK)],
                osems[bb],
            )
    for cp in wb_cps.values():
        cp.wait()


@jax.jit
def _run(idx, table_flat):
    k = pl.kernel(
        _body,
        out_type=jax.ShapeDtypeStruct((_TOTAL, _D), jnp.float32),
        mesh=plsc.VectorSubcoreMesh(core_axis_name="c", subcore_axis_name="s"),
        scratch_types=[
            pltpu.VMEM((_IDXROWS_W, _IDX_W), jnp.int32),
            pltpu.VMEM((_NBUF, _CHUNK, _D), jnp.float32),
            [pltpu.SemaphoreType.DMA] * _NBUF,
            [pltpu.SemaphoreType.DMA] * _NBUF,
        ],
        compiler_params=pltpu.CompilerParams(use_tc_tiling_on_sc=False),
    )
    return k(idx, table_flat)


def kernel(x, tables):
    offs = (jnp.arange(_NUM_FIELDS, dtype=jnp.int32) * _VOCAB)[:, None]
    idx = (x.T + offs).reshape(_TOTAL // _IDX_W, _IDX_W)
    out = _run(idx, tables.reshape(_NUM_FIELDS * _VOCAB, _D))
    return out.reshape(_NUM_FIELDS, _BATCH, _D)
